# baseline (device time: 99323 ns/iter reference)
import jax
import jax.numpy as jnp
from jax import lax
from jax.experimental import pallas as pl
from jax.experimental.pallas import tpu as pltpu

T = 1024
D = 2048
V_SHARD = 16384
V_CHUNK = 2048
N_CHUNKS = V_SHARD // V_CHUNK


def kernel(x, W, labels):
    labels2d = labels.reshape(T, 1)

    def body(x_ref, w_ref, lab_ref, out_ref,
             xb_ref, s_ref, ll_ref, comm_ref, send_sem, recv_sem):
        j = pl.program_id(0)
        my_x = lax.axis_index("x")
        my_y = lax.axis_index("y")
        my_z = lax.axis_index("z")

        @pl.when(j == 0)
        def _():
            barrier_sem = pltpu.get_barrier_semaphore()
            pl.semaphore_signal(
                barrier_sem, inc=1,
                device_id=(my_x, 1 - my_y, my_z),
                device_id_type=pl.DeviceIdType.MESH,
            )
            xb_ref[:, :] = x_ref[:, :].astype(jnp.bfloat16)
            s_ref[:, :] = jnp.zeros((T, 1), jnp.float32)
            ll_ref[:, :] = jnp.zeros((T, 1), jnp.float32)

        chunk = jnp.dot(xb_ref[:, :], w_ref[:, :].astype(jnp.bfloat16),
                        preferred_element_type=jnp.float32)
        s_ref[:, :] += jnp.sum(jnp.exp(chunk), axis=1, keepdims=True)
        local_lab = lab_ref[:, :] - my_y * V_SHARD - j * V_CHUNK
        col = lax.broadcasted_iota(jnp.int32, (T, V_CHUNK), 1)
        ll_ref[:, :] += jnp.sum(
            jnp.where(col == local_lab, chunk, 0.0), axis=1, keepdims=True)

        @pl.when(j == N_CHUNKS - 1)
        def _():
            comm_ref[0, :, 0:1] = s_ref[:, :]
            comm_ref[0, :, 1:2] = ll_ref[:, :]
            pl.semaphore_wait(pltpu.get_barrier_semaphore(), 1)
            rdma = pltpu.make_async_remote_copy(
                src_ref=comm_ref.at[0],
                dst_ref=comm_ref.at[1],
                send_sem=send_sem,
                recv_sem=recv_sem,
                device_id=(my_x, 1 - my_y, my_z),
                device_id_type=pl.DeviceIdType.MESH,
            )
            rdma.start()
            rdma.wait()

            ps = comm_ref[1, :, 0:1]
            pll = comm_ref[1, :, 1:2]
            out_ref[:, :] = (jnp.log(s_ref[:, :] + ps)
                             - (ll_ref[:, :] + pll))

    nll = pl.pallas_call(
        body,
        grid=(N_CHUNKS,),
        in_specs=[
            pl.BlockSpec((T, D), lambda j: (0, 0)),
            pl.BlockSpec((D, V_CHUNK), lambda j: (0, j)),
            pl.BlockSpec((T, 1), lambda j: (0, 0)),
        ],
        out_specs=pl.BlockSpec((T, 1), lambda j: (0, 0)),
        out_shape=jax.ShapeDtypeStruct((T, 1), jnp.float32),
        scratch_shapes=[
            pltpu.VMEM((T, D), jnp.bfloat16),
            pltpu.VMEM((T, 1), jnp.float32),
            pltpu.VMEM((T, 1), jnp.float32),
            pltpu.VMEM((2, T, 4), jnp.float32),
            pltpu.SemaphoreType.DMA,
            pltpu.SemaphoreType.DMA,
        ],
        compiler_params=pltpu.CompilerParams(
            dimension_semantics=("arbitrary",),
            collective_id=0,
            vmem_limit_bytes=60 * 1024 * 1024,
        ),
    )(x, W, labels2d)
    return nll.reshape(T)


# device time: 93468 ns/iter; 1.0626x vs baseline; 1.0626x over previous
import jax
import jax.numpy as jnp
from jax import lax
from jax.experimental import pallas as pl
from jax.experimental.pallas import tpu as pltpu

T = 1024
D = 2048
V_SHARD = 16384
V_CHUNK = 2048
N_CHUNKS = V_SHARD // V_CHUNK


def kernel(x, W, labels):
    labels2d = labels.reshape(T, 1)

    def body(x_ref, w_ref, lab_ref, out_ref,
             xb_ref, s_ref, ll_ref, comm_ref, send_sem, recv_sem):
        j = pl.program_id(0)
        my_x = lax.axis_index("x")
        my_y = lax.axis_index("y")
        my_z = lax.axis_index("z")

        @pl.when(j == 0)
        def _():
            barrier_sem = pltpu.get_barrier_semaphore()
            pl.semaphore_signal(
                barrier_sem, inc=1,
                device_id=(my_x, 1 - my_y, my_z),
                device_id_type=pl.DeviceIdType.MESH,
            )
            xb_ref[:, :] = x_ref[:, :].astype(jnp.bfloat16)
            s_ref[:, :] = jnp.zeros((T, 1), jnp.float32)
            ll_ref[:, :] = jnp.zeros((T, 1), jnp.float32)

        chunk = jnp.dot(xb_ref[:, :], w_ref[:, :].astype(jnp.bfloat16),
                        preferred_element_type=jnp.float32)
        s_ref[:, :] += jnp.sum(jnp.exp(chunk), axis=1, keepdims=True)
        local_lab = lab_ref[:, :] - my_y * V_SHARD - j * V_CHUNK
        col = lax.broadcasted_iota(jnp.int32, (T, V_CHUNK), 1)
        ll_ref[:, :] += jnp.sum(
            jnp.where(col == local_lab, chunk, 0.0), axis=1, keepdims=True)

        @pl.when(j == N_CHUNKS - 1)
        def _():
            s8 = s_ref[:, :].reshape(8, 128)
            ll8 = ll_ref[:, :].reshape(8, 128)
            comm_ref[0, 0:8, :] = s8
            comm_ref[0, 8:16, :] = ll8
            pl.semaphore_wait(pltpu.get_barrier_semaphore(), 1)
            rdma = pltpu.make_async_remote_copy(
                src_ref=comm_ref.at[0],
                dst_ref=comm_ref.at[1],
                send_sem=send_sem,
                recv_sem=recv_sem,
                device_id=(my_x, 1 - my_y, my_z),
                device_id_type=pl.DeviceIdType.MESH,
            )
            rdma.start()
            rdma.wait()

            ps8 = comm_ref[1, 0:8, :]
            pll8 = comm_ref[1, 8:16, :]
            out_ref[:, :] = jnp.log(s8 + ps8) - (ll8 + pll8)

    nll8 = pl.pallas_call(
        body,
        grid=(N_CHUNKS,),
        in_specs=[
            pl.BlockSpec((T, D), lambda j: (0, 0)),
            pl.BlockSpec((D, V_CHUNK), lambda j: (0, j)),
            pl.BlockSpec((T, 1), lambda j: (0, 0)),
        ],
        out_specs=pl.BlockSpec((8, 128), lambda j: (0, 0)),
        out_shape=jax.ShapeDtypeStruct((8, 128), jnp.float32),
        scratch_shapes=[
            pltpu.VMEM((T, D), jnp.bfloat16),
            pltpu.VMEM((T, 1), jnp.float32),
            pltpu.VMEM((T, 1), jnp.float32),
            pltpu.VMEM((2, 16, 128), jnp.float32),
            pltpu.SemaphoreType.DMA,
            pltpu.SemaphoreType.DMA,
        ],
        compiler_params=pltpu.CompilerParams(
            dimension_semantics=("arbitrary",),
            collective_id=0,
            vmem_limit_bytes=60 * 1024 * 1024,
        ),
    )(x, W, labels2d)
    return nll8.reshape(T)
